# baseline (device time: 265244 ns/iter reference)
import jax
import jax.numpy as jnp
from jax import lax
from jax.experimental import pallas as pl
from jax.experimental.pallas import tpu as pltpu

N_DEV = 4


def kernel(Q, K, V):
    B, QL, H, D = Q.shape
    S = K.shape[1]
    HD = H * D
    scale = D ** -0.5

    q2 = Q.reshape(B, QL, HD)
    k2 = K.reshape(B, S, HD)
    v2 = V.reshape(B, S, HD)

    def compute_body(q_ref, k_ref, v_ref, o_ref, l_ref):
        q = q_ref[0]
        k = k_ref[0]
        v = v_ref[0]
        s = lax.dot_general(
            q, k, (((1,), (1,)), ((), ())),
            preferred_element_type=jnp.float32,
        ) * scale
        p = jnp.exp(s)
        l = jnp.sum(p, axis=1)
        o = lax.dot_general(
            p, v, (((1,), (0,)), ((), ())),
            preferred_element_type=jnp.float32,
        )
        o_ref[0] = o
        l_ref[0] = jnp.broadcast_to(l[:, None], (QL, D))

    o_part, l_full = pl.pallas_call(
        compute_body,
        grid=(B, H),
        in_specs=[
            pl.BlockSpec((1, QL, D), lambda b, h: (b, 0, h)),
            pl.BlockSpec((1, S, D), lambda b, h: (b, 0, h)),
            pl.BlockSpec((1, S, D), lambda b, h: (b, 0, h)),
        ],
        out_specs=[
            pl.BlockSpec((1, QL, D), lambda b, h: (b, 0, h)),
            pl.BlockSpec((1, QL, D), lambda b, h: (b, 0, h)),
        ],
        out_shape=[
            jax.ShapeDtypeStruct((B, QL, HD), jnp.float32),
            jax.ShapeDtypeStruct((B, QL, HD), jnp.float32),
        ],
    )(q2, k2, v2)

    l_part = l_full[:, :, ::D]

    def ring_body(o_ref, l_ref, o_out, l_out, co, cl,
                  o_send, o_recv, l_send, l_recv):
        my = lax.axis_index("i")

        barrier = pltpu.get_barrier_semaphore()
        for j in range(1, N_DEV):
            peer = lax.rem(my + j, N_DEV)
            pl.semaphore_signal(
                barrier, inc=1,
                device_id=(peer,), device_id_type=pl.DeviceIdType.MESH,
            )
        pl.semaphore_wait(barrier, N_DEV - 1)

        loc_o = pltpu.make_async_copy(o_ref, co.at[0], o_send.at[0])
        loc_l = pltpu.make_async_copy(l_ref, cl.at[0], l_send.at[0])
        loc_o.start()
        loc_l.start()

        sends = []
        for j in range(1, N_DEV):
            peer = lax.rem(my + j, N_DEV)
            slot = N_DEV - j
            so = pltpu.make_async_remote_copy(
                src_ref=o_ref, dst_ref=co.at[slot],
                send_sem=o_send.at[j], recv_sem=o_recv.at[slot],
                device_id=(peer,), device_id_type=pl.DeviceIdType.MESH,
            )
            sl = pltpu.make_async_remote_copy(
                src_ref=l_ref, dst_ref=cl.at[slot],
                send_sem=l_send.at[j], recv_sem=l_recv.at[slot],
                device_id=(peer,), device_id_type=pl.DeviceIdType.MESH,
            )
            so.start()
            sl.start()
            sends.append((so, sl))

        for s in range(1, N_DEV):
            recv_o = pltpu.make_async_remote_copy(
                src_ref=o_ref, dst_ref=co.at[s],
                send_sem=o_send.at[0], recv_sem=o_recv.at[s],
                device_id=(my,), device_id_type=pl.DeviceIdType.MESH,
            )
            recv_l = pltpu.make_async_remote_copy(
                src_ref=l_ref, dst_ref=cl.at[s],
                send_sem=l_send.at[0], recv_sem=l_recv.at[s],
                device_id=(my,), device_id_type=pl.DeviceIdType.MESH,
            )
            recv_o.wait_recv()
            recv_l.wait_recv()

        loc_o.wait()
        loc_l.wait()

        o_out[...] = co[0] + co[1] + co[2] + co[3]
        l_out[...] = cl[0] + cl[1] + cl[2] + cl[3]

        for so, sl in sends:
            so.wait_send()
            sl.wait_send()

    o_sum, l_sum = pl.pallas_call(
        ring_body,
        in_specs=[
            pl.BlockSpec(memory_space=pltpu.VMEM),
            pl.BlockSpec(memory_space=pltpu.VMEM),
        ],
        out_specs=[
            pl.BlockSpec(memory_space=pltpu.VMEM),
            pl.BlockSpec(memory_space=pltpu.VMEM),
        ],
        out_shape=[
            jax.ShapeDtypeStruct((B, QL, HD), jnp.float32),
            jax.ShapeDtypeStruct((B, QL, H), jnp.float32),
        ],
        scratch_shapes=[
            pltpu.VMEM((N_DEV, B, QL, HD), jnp.float32),
            pltpu.VMEM((N_DEV, B, QL, H), jnp.float32),
            pltpu.SemaphoreType.DMA((N_DEV,)),
            pltpu.SemaphoreType.DMA((N_DEV,)),
            pltpu.SemaphoreType.DMA((N_DEV,)),
            pltpu.SemaphoreType.DMA((N_DEV,)),
        ],
        compiler_params=pltpu.CompilerParams(collective_id=0),
    )(o_part, l_part)

    return (o_sum.reshape(B, QL, H, D) / l_sum[..., None]).astype(jnp.float32)


# device time: 137150 ns/iter; 1.9340x vs baseline; 1.9340x over previous
import jax
import jax.numpy as jnp
from jax import lax
from jax.experimental import pallas as pl
from jax.experimental.pallas import tpu as pltpu

N_DEV = 4
LW = 8


def kernel(Q, K, V):
    B, QL, H, D = Q.shape
    S = K.shape[1]
    scale = D ** -0.5

    GRID = B * H

    def compute_body(q_hbm, k_hbm, v_hbm, o_hbm, l_hbm,
                     qbuf, kbuf, vbuf, obuf, lbuf,
                     qsem, ksem, vsem, osem, lsem):
        b = pl.program_id(0)
        h = pl.program_id(1)
        i = b * H + h
        slot = lax.rem(i, 2)
        nslot = lax.rem(i + 1, 2)

        def in_copies(bb, hh, sl):
            return (
                pltpu.make_async_copy(
                    q_hbm.at[bb, :, hh, :], qbuf.at[sl], qsem.at[sl]),
                pltpu.make_async_copy(
                    k_hbm.at[bb, :, hh, :], kbuf.at[sl], ksem.at[sl]),
                pltpu.make_async_copy(
                    v_hbm.at[bb, :, hh, :], vbuf.at[sl], vsem.at[sl]),
            )

        @pl.when(i == 0)
        def _():
            for c in in_copies(b, h, slot):
                c.start()

        @pl.when(i + 1 < GRID)
        def _():
            ni = i + 1
            for c in in_copies(lax.div(ni, H), lax.rem(ni, H), nslot):
                c.start()

        for c in in_copies(b, h, slot):
            c.wait()

        q = qbuf[slot]
        k = kbuf[slot]
        v = vbuf[slot]
        s = lax.dot_general(
            q, k, (((1,), (1,)), ((), ())),
            preferred_element_type=jnp.float32,
        ) * scale
        p = jnp.exp(s)
        l = jnp.sum(p, axis=1)
        o = lax.dot_general(
            p, v, (((1,), (0,)), ((), ())),
            preferred_element_type=jnp.float32,
        )

        def out_copies(sl):
            return (
                pltpu.make_async_copy(
                    obuf.at[sl], o_hbm.at[b, :, h, :], osem.at[sl]),
                pltpu.make_async_copy(
                    lbuf.at[sl], l_hbm.at[b, :, h, :], lsem.at[sl]),
            )

        @pl.when(i >= 2)
        def _():
            for c in out_copies(slot):
                c.wait()

        obuf[slot] = o
        lbuf[slot] = jnp.broadcast_to(l[:, None], (QL, LW))
        for c in out_copies(slot):
            c.start()

        @pl.when(i == GRID - 1)
        def _():
            for c in out_copies(slot):
                c.wait()
            for c in out_copies(nslot):
                c.wait()

    o_part, l_part = pl.pallas_call(
        compute_body,
        grid=(B, H),
        in_specs=[pl.BlockSpec(memory_space=pl.ANY)] * 3,
        out_specs=[pl.BlockSpec(memory_space=pl.ANY)] * 2,
        out_shape=[
            jax.ShapeDtypeStruct((B, QL, H, D), jnp.float32),
            jax.ShapeDtypeStruct((B, QL, H, LW), jnp.float32),
        ],
        scratch_shapes=[
            pltpu.VMEM((2, QL, D), jnp.float32),
            pltpu.VMEM((2, S, D), jnp.float32),
            pltpu.VMEM((2, S, D), jnp.float32),
            pltpu.VMEM((2, QL, D), jnp.float32),
            pltpu.VMEM((2, QL, LW), jnp.float32),
            pltpu.SemaphoreType.DMA((2,)),
            pltpu.SemaphoreType.DMA((2,)),
            pltpu.SemaphoreType.DMA((2,)),
            pltpu.SemaphoreType.DMA((2,)),
            pltpu.SemaphoreType.DMA((2,)),
        ],
    )(Q, K, V)

    def ring_body(o_ref, l_ref, o_out, l_out, co, cl,
                  o_send, o_recv, l_send, l_recv):
        my = lax.axis_index("i")

        barrier = pltpu.get_barrier_semaphore()
        for j in range(1, N_DEV):
            peer = lax.rem(my + j, N_DEV)
            pl.semaphore_signal(
                barrier, inc=1,
                device_id=(peer,), device_id_type=pl.DeviceIdType.MESH,
            )
        pl.semaphore_wait(barrier, N_DEV - 1)

        loc_o = pltpu.make_async_copy(o_ref, co.at[0], o_send.at[0])
        loc_l = pltpu.make_async_copy(l_ref, cl.at[0], l_send.at[0])
        loc_o.start()
        loc_l.start()

        sends = []
        for j in range(1, N_DEV):
            peer = lax.rem(my + j, N_DEV)
            slot = N_DEV - j
            so = pltpu.make_async_remote_copy(
                src_ref=o_ref, dst_ref=co.at[slot],
                send_sem=o_send.at[j], recv_sem=o_recv.at[slot],
                device_id=(peer,), device_id_type=pl.DeviceIdType.MESH,
            )
            sl = pltpu.make_async_remote_copy(
                src_ref=l_ref, dst_ref=cl.at[slot],
                send_sem=l_send.at[j], recv_sem=l_recv.at[slot],
                device_id=(peer,), device_id_type=pl.DeviceIdType.MESH,
            )
            so.start()
            sl.start()
            sends.append((so, sl))

        for s in range(1, N_DEV):
            recv_o = pltpu.make_async_remote_copy(
                src_ref=o_ref, dst_ref=co.at[s],
                send_sem=o_send.at[0], recv_sem=o_recv.at[s],
                device_id=(my,), device_id_type=pl.DeviceIdType.MESH,
            )
            recv_l = pltpu.make_async_remote_copy(
                src_ref=l_ref, dst_ref=cl.at[s],
                send_sem=l_send.at[0], recv_sem=l_recv.at[s],
                device_id=(my,), device_id_type=pl.DeviceIdType.MESH,
            )
            recv_o.wait_recv()
            recv_l.wait_recv()

        loc_o.wait()
        loc_l.wait()

        o_out[...] = co[0] + co[1] + co[2] + co[3]
        l_out[...] = cl[0] + cl[1] + cl[2] + cl[3]

        for so, sl in sends:
            so.wait_send()
            sl.wait_send()

    o_sum, l_sum = pl.pallas_call(
        ring_body,
        in_specs=[
            pl.BlockSpec(memory_space=pltpu.VMEM),
            pl.BlockSpec(memory_space=pltpu.VMEM),
        ],
        out_specs=[
            pl.BlockSpec(memory_space=pltpu.VMEM),
            pl.BlockSpec(memory_space=pltpu.VMEM),
        ],
        out_shape=[
            jax.ShapeDtypeStruct((B, QL, H, D), jnp.float32),
            jax.ShapeDtypeStruct((B, QL, H, LW), jnp.float32),
        ],
        scratch_shapes=[
            pltpu.VMEM((N_DEV, B, QL, H, D), jnp.float32),
            pltpu.VMEM((N_DEV, B, QL, H, LW), jnp.float32),
            pltpu.SemaphoreType.DMA((N_DEV,)),
            pltpu.SemaphoreType.DMA((N_DEV,)),
            pltpu.SemaphoreType.DMA((N_DEV,)),
            pltpu.SemaphoreType.DMA((N_DEV,)),
        ],
        compiler_params=pltpu.CompilerParams(collective_id=0),
    )(o_part, l_part)

    return o_sum / l_sum[..., 0:1]


# device time: 75421 ns/iter; 3.5168x vs baseline; 1.8185x over previous
import jax
import jax.numpy as jnp
from jax import lax
from jax.experimental import pallas as pl
from jax.experimental.pallas import tpu as pltpu

N_DEV = 4
LW = 8


def kernel(Q, K, V):
    B, QL, H, D = Q.shape
    S = K.shape[1]
    scale = D ** -0.5

    def compute_body(q_hbm, k_hbm, v_hbm, o_hbm, l_hbm,
                     qbuf, kbuf, vbuf, obuf, lbuf,
                     qsem, ksem, vsem, osem, lsem):
        h = pl.program_id(0)
        slot = lax.rem(h, 2)
        nslot = lax.rem(h + 1, 2)

        def in_copies(hh, sl):
            return (
                pltpu.make_async_copy(
                    q_hbm.at[:, :, hh, :], qbuf.at[sl], qsem.at[sl]),
                pltpu.make_async_copy(
                    k_hbm.at[:, :, hh, :], kbuf.at[sl], ksem.at[sl]),
                pltpu.make_async_copy(
                    v_hbm.at[:, :, hh, :], vbuf.at[sl], vsem.at[sl]),
            )

        def out_copies(hh, sl):
            return (
                pltpu.make_async_copy(
                    obuf.at[sl], o_hbm.at[:, :, hh, :], osem.at[sl]),
                pltpu.make_async_copy(
                    lbuf.at[sl], l_hbm.at[:, :, hh, :], lsem.at[sl]),
            )

        @pl.when(h == 0)
        def _():
            for c in in_copies(h, slot):
                c.start()

        @pl.when(h + 1 < H)
        def _():
            for c in in_copies(h + 1, nslot):
                c.start()

        for c in in_copies(h, slot):
            c.wait()

        @pl.when(h >= 2)
        def _():
            for c in out_copies(h, slot):
                c.wait()

        for b in range(B):
            q = qbuf[slot, b]
            k = kbuf[slot, b]
            v = vbuf[slot, b]
            s = lax.dot_general(
                q, k, (((1,), (1,)), ((), ())),
                preferred_element_type=jnp.float32,
            ) * scale
            p = jnp.exp(s)
            l = jnp.sum(p, axis=1)
            o = lax.dot_general(
                p, v, (((1,), (0,)), ((), ())),
                preferred_element_type=jnp.float32,
            )
            obuf[slot, b] = o
            lbuf[slot, b] = jnp.broadcast_to(l[:, None], (QL, LW))

        for c in out_copies(h, slot):
            c.start()

        @pl.when(h == H - 1)
        def _():
            for sl in (slot, nslot):
                for c in out_copies(h, sl):
                    c.wait()

    o_part, l_part = pl.pallas_call(
        compute_body,
        grid=(H,),
        in_specs=[pl.BlockSpec(memory_space=pl.ANY)] * 3,
        out_specs=[pl.BlockSpec(memory_space=pl.ANY)] * 2,
        out_shape=[
            jax.ShapeDtypeStruct((B, QL, H, D), jnp.float32),
            jax.ShapeDtypeStruct((B, QL, H, LW), jnp.float32),
        ],
        scratch_shapes=[
            pltpu.VMEM((2, B, QL, D), jnp.float32),
            pltpu.VMEM((2, B, S, D), jnp.float32),
            pltpu.VMEM((2, B, S, D), jnp.float32),
            pltpu.VMEM((2, B, QL, D), jnp.float32),
            pltpu.VMEM((2, B, QL, LW), jnp.float32),
            pltpu.SemaphoreType.DMA((2,)),
            pltpu.SemaphoreType.DMA((2,)),
            pltpu.SemaphoreType.DMA((2,)),
            pltpu.SemaphoreType.DMA((2,)),
            pltpu.SemaphoreType.DMA((2,)),
        ],
    )(Q, K, V)

    def ring_body(o_ref, l_ref, o_out, l_out, co, cl,
                  o_send, o_recv, l_send, l_recv):
        my = lax.axis_index("i")

        barrier = pltpu.get_barrier_semaphore()
        for j in range(1, N_DEV):
            peer = lax.rem(my + j, N_DEV)
            pl.semaphore_signal(
                barrier, inc=1,
                device_id=(peer,), device_id_type=pl.DeviceIdType.MESH,
            )
        pl.semaphore_wait(barrier, N_DEV - 1)

        loc_o = pltpu.make_async_copy(o_ref, co.at[0], o_send.at[0])
        loc_l = pltpu.make_async_copy(l_ref, cl.at[0], l_send.at[0])
        loc_o.start()
        loc_l.start()

        sends = []
        for j in range(1, N_DEV):
            peer = lax.rem(my + j, N_DEV)
            slot = N_DEV - j
            so = pltpu.make_async_remote_copy(
                src_ref=o_ref, dst_ref=co.at[slot],
                send_sem=o_send.at[j], recv_sem=o_recv.at[slot],
                device_id=(peer,), device_id_type=pl.DeviceIdType.MESH,
            )
            sl = pltpu.make_async_remote_copy(
                src_ref=l_ref, dst_ref=cl.at[slot],
                send_sem=l_send.at[j], recv_sem=l_recv.at[slot],
                device_id=(peer,), device_id_type=pl.DeviceIdType.MESH,
            )
            so.start()
            sl.start()
            sends.append((so, sl))

        for s in range(1, N_DEV):
            recv_o = pltpu.make_async_remote_copy(
                src_ref=o_ref, dst_ref=co.at[s],
                send_sem=o_send.at[0], recv_sem=o_recv.at[s],
                device_id=(my,), device_id_type=pl.DeviceIdType.MESH,
            )
            recv_l = pltpu.make_async_remote_copy(
                src_ref=l_ref, dst_ref=cl.at[s],
                send_sem=l_send.at[0], recv_sem=l_recv.at[s],
                device_id=(my,), device_id_type=pl.DeviceIdType.MESH,
            )
            recv_o.wait_recv()
            recv_l.wait_recv()

        loc_o.wait()
        loc_l.wait()

        o_out[...] = co[0] + co[1] + co[2] + co[3]
        l_out[...] = cl[0] + cl[1] + cl[2] + cl[3]

        for so, sl in sends:
            so.wait_send()
            sl.wait_send()

    o_sum, l_sum = pl.pallas_call(
        ring_body,
        in_specs=[
            pl.BlockSpec(memory_space=pltpu.VMEM),
            pl.BlockSpec(memory_space=pltpu.VMEM),
        ],
        out_specs=[
            pl.BlockSpec(memory_space=pltpu.VMEM),
            pl.BlockSpec(memory_space=pltpu.VMEM),
        ],
        out_shape=[
            jax.ShapeDtypeStruct((B, QL, H, D), jnp.float32),
            jax.ShapeDtypeStruct((B, QL, H, LW), jnp.float32),
        ],
        scratch_shapes=[
            pltpu.VMEM((N_DEV, B, QL, H, D), jnp.float32),
            pltpu.VMEM((N_DEV, B, QL, H, LW), jnp.float32),
            pltpu.SemaphoreType.DMA((N_DEV,)),
            pltpu.SemaphoreType.DMA((N_DEV,)),
            pltpu.SemaphoreType.DMA((N_DEV,)),
            pltpu.SemaphoreType.DMA((N_DEV,)),
        ],
        compiler_params=pltpu.CompilerParams(collective_id=0),
    )(o_part, l_part)

    return o_sum / l_sum[..., 0:1]


# device time: 75116 ns/iter; 3.5311x vs baseline; 1.0041x over previous
import jax
import jax.numpy as jnp
from jax import lax
from jax.experimental import pallas as pl
from jax.experimental.pallas import tpu as pltpu

N_DEV = 4
LW = 8


def kernel(Q, K, V):
    B, QL, H, D = Q.shape
    S = K.shape[1]
    scale = D ** -0.5

    def compute_body(q_hbm, k_hbm, v_hbm, o_hbm, l_hbm,
                     qbuf, kbuf, vbuf, obuf, lbuf,
                     qsem, ksem, vsem, osem, lsem):
        h = pl.program_id(0)
        slot = lax.rem(h, 2)
        nslot = lax.rem(h + 1, 2)

        def in_copies(hh, sl):
            return (
                pltpu.make_async_copy(
                    q_hbm.at[:, :, hh, :], qbuf.at[sl], qsem.at[sl]),
                pltpu.make_async_copy(
                    k_hbm.at[:, :, hh, :], kbuf.at[sl], ksem.at[sl]),
                pltpu.make_async_copy(
                    v_hbm.at[:, :, hh, :], vbuf.at[sl], vsem.at[sl]),
            )

        def out_copies(hh, sl):
            return (
                pltpu.make_async_copy(
                    obuf.at[sl], o_hbm.at[:, :, hh, :], osem.at[sl]),
                pltpu.make_async_copy(
                    lbuf.at[sl], l_hbm.at[:, :, hh, :], lsem.at[sl]),
            )

        @pl.when(h == 0)
        def _():
            for c in in_copies(h, slot):
                c.start()

        @pl.when(h + 1 < H)
        def _():
            for c in in_copies(h + 1, nslot):
                c.start()

        for c in in_copies(h, slot):
            c.wait()

        @pl.when(h >= 2)
        def _():
            for c in out_copies(h, slot):
                c.wait()

        for b in range(B):
            q = qbuf[slot, b]
            k = kbuf[slot, b]
            v = vbuf[slot, b]
            s = lax.dot_general(
                q, k, (((1,), (1,)), ((), ())),
                preferred_element_type=jnp.float32,
            ) * scale
            p = jnp.exp(s)
            l = jnp.sum(p, axis=1)
            o = lax.dot_general(
                p, v, (((1,), (0,)), ((), ())),
                preferred_element_type=jnp.float32,
            )
            obuf[slot, b] = o
            lbuf[slot, b] = jnp.broadcast_to(l[:, None], (QL, LW))

        for c in out_copies(h, slot):
            c.start()

        @pl.when(h == H - 1)
        def _():
            for sl in (slot, nslot):
                for c in out_copies(h, sl):
                    c.wait()

    o_part, l_part = pl.pallas_call(
        compute_body,
        grid=(H,),
        in_specs=[pl.BlockSpec(memory_space=pl.ANY)] * 3,
        out_specs=[pl.BlockSpec(memory_space=pl.ANY)] * 2,
        out_shape=[
            jax.ShapeDtypeStruct((B, QL, H, D), jnp.float32),
            jax.ShapeDtypeStruct((B, QL, H, LW), jnp.float32),
        ],
        scratch_shapes=[
            pltpu.VMEM((2, B, QL, D), jnp.float32),
            pltpu.VMEM((2, B, S, D), jnp.float32),
            pltpu.VMEM((2, B, S, D), jnp.float32),
            pltpu.VMEM((2, B, QL, D), jnp.float32),
            pltpu.VMEM((2, B, QL, LW), jnp.float32),
            pltpu.SemaphoreType.DMA((2,)),
            pltpu.SemaphoreType.DMA((2,)),
            pltpu.SemaphoreType.DMA((2,)),
            pltpu.SemaphoreType.DMA((2,)),
            pltpu.SemaphoreType.DMA((2,)),
        ],
    )(Q, K, V)

    def ring_body(o_ref, l_ref, o_out, l_out, co, cl,
                  o_send, o_recv, l_send, l_recv):
        my = lax.axis_index("i")

        barrier = pltpu.get_barrier_semaphore()
        for j in range(1, N_DEV):
            peer = lax.rem(my + j, N_DEV)
            pl.semaphore_signal(
                barrier, inc=1,
                device_id=(peer,), device_id_type=pl.DeviceIdType.MESH,
            )
        pl.semaphore_wait(barrier, N_DEV - 1)

        sends = []
        for j in range(1, N_DEV):
            peer = lax.rem(my + j, N_DEV)
            slot = N_DEV - j
            so = pltpu.make_async_remote_copy(
                src_ref=o_ref, dst_ref=co.at[slot],
                send_sem=o_send.at[j], recv_sem=o_recv.at[slot],
                device_id=(peer,), device_id_type=pl.DeviceIdType.MESH,
            )
            sl = pltpu.make_async_remote_copy(
                src_ref=l_ref, dst_ref=cl.at[slot],
                send_sem=l_send.at[j], recv_sem=l_recv.at[slot],
                device_id=(peer,), device_id_type=pl.DeviceIdType.MESH,
            )
            so.start()
            sl.start()
            sends.append((so, sl))

        o_out[...] = o_ref[...]
        l_out[...] = l_ref[...]
        for s in (1, 3, 2):
            recv_o = pltpu.make_async_remote_copy(
                src_ref=o_ref, dst_ref=co.at[s],
                send_sem=o_send.at[0], recv_sem=o_recv.at[s],
                device_id=(my,), device_id_type=pl.DeviceIdType.MESH,
            )
            recv_l = pltpu.make_async_remote_copy(
                src_ref=l_ref, dst_ref=cl.at[s],
                send_sem=l_send.at[0], recv_sem=l_recv.at[s],
                device_id=(my,), device_id_type=pl.DeviceIdType.MESH,
            )
            recv_o.wait_recv()
            recv_l.wait_recv()
            o_out[...] += co[s]
            l_out[...] += cl[s]

        for so, sl in sends:
            so.wait_send()
            sl.wait_send()

    o_sum, l_sum = pl.pallas_call(
        ring_body,
        in_specs=[
            pl.BlockSpec(memory_space=pltpu.VMEM),
            pl.BlockSpec(memory_space=pltpu.VMEM),
        ],
        out_specs=[
            pl.BlockSpec(memory_space=pltpu.VMEM),
            pl.BlockSpec(memory_space=pltpu.VMEM),
        ],
        out_shape=[
            jax.ShapeDtypeStruct((B, QL, H, D), jnp.float32),
            jax.ShapeDtypeStruct((B, QL, H, LW), jnp.float32),
        ],
        scratch_shapes=[
            pltpu.VMEM((N_DEV, B, QL, H, D), jnp.float32),
            pltpu.VMEM((N_DEV, B, QL, H, LW), jnp.float32),
            pltpu.SemaphoreType.DMA((N_DEV,)),
            pltpu.SemaphoreType.DMA((N_DEV,)),
            pltpu.SemaphoreType.DMA((N_DEV,)),
            pltpu.SemaphoreType.DMA((N_DEV,)),
        ],
        compiler_params=pltpu.CompilerParams(collective_id=0),
    )(o_part, l_part)

    return o_sum / l_sum[..., 0:1]
